# Initial kernel scaffold; baseline (speedup 1.0000x reference)
import jax
import jax.numpy as jnp
from jax.experimental import pallas as pl


def kernel(x, edge_index, edge_attr, node_degree, weights, readout_weights):
    raise NotImplementedError("write your pallas kernel here")


# speedup vs baseline: 2.5247x; 2.5247x over previous
import functools

import jax
import jax.numpy as jnp
from jax import lax
from jax.experimental import pallas as pl
from jax.experimental.pallas import tpu as pltpu
from jax.experimental.pallas import tpu_sc as plsc

_N = 10000
_E = 320000
_NV = 128
_NE = 16
_MAXD = 32
_MIND = 1
_T = 4
_NOUT = 10
_B = _MAXD - _MIND + 1

_NC = 2
_NS = 16
_NW = _NC * _NS
_CHUNK = 128
_CPW = 80
_HALF = _CPW // 2
_EPAD = _NW * _CPW * _CHUNK
_NCHUNKS = _EPAD // _CHUNK
_NPAD = 10112
_RPT = _NPAD // _NS

_R = 1000
_NBLK = _N // _R


def _zero_acc(buf_v, acc_sh, sid, width):
    def zrow(r, carry):
        for c in range(width // 16):
            buf_v[r, pl.ds(c * 16, 16)] = jnp.zeros((16,), jnp.float32)
        return carry

    lax.fori_loop(0, _CHUNK, zrow, 0)
    for k in range(0, _RPT, _CHUNK):
        rows = min(_CHUNK, _RPT - k)
        pltpu.sync_copy(buf_v.at[pl.ds(0, rows)],
                        acc_sh.at[pl.ds(sid * _RPT + k, rows)])


def _sc_x_body(h_hbm, src_hbm, dst_hbm, outx_hbm,
               src_v, dst_v, rows_v, accx_sh, sem):
    cid = lax.axis_index("c")
    sid = lax.axis_index("s")
    w = sid * _NC + cid

    _zero_acc(rows_v, accx_sh, sid, _NV)
    plsc.subcore_barrier()

    def step(j, carry):
        pltpu.async_copy(h_hbm.at[src_v.at[j]], rows_v, sem).wait()
        pltpu.sync_copy(rows_v, accx_sh.at[dst_v.at[j]], add=True)
        return carry

    for hh in range(2):
        half_base = w * _CPW + hh * _HALF
        pltpu.sync_copy(src_hbm.at[pl.ds(half_base, _HALF)], src_v)
        pltpu.sync_copy(dst_hbm.at[pl.ds(half_base, _HALF)], dst_v)
        lax.fori_loop(0, _HALF, step, 0)
    plsc.subcore_barrier()

    pltpu.sync_copy(accx_sh.at[pl.ds(sid * _RPT, _RPT)],
                    outx_hbm.at[cid, pl.ds(sid * _RPT, _RPT)])


def _sc_e_body(ea_hbm, dst_hbm, oute_hbm, dst_v, erow_v, acce_sh):
    cid = lax.axis_index("c")
    sid = lax.axis_index("s")
    w = sid * _NC + cid

    _zero_acc(erow_v, acce_sh, sid, _NE)
    plsc.subcore_barrier()

    def step(j, carry):
        base = (w * _CPW + j) * _CHUNK
        pltpu.sync_copy(ea_hbm.at[pl.ds(base, _CHUNK)], erow_v)
        pltpu.sync_copy(erow_v, acce_sh.at[dst_v.at[j]], add=True)
        return carry

    pltpu.sync_copy(dst_hbm.at[pl.ds(w * _CPW, _CPW)], dst_v)
    lax.fori_loop(0, _CPW, step, 0)
    plsc.subcore_barrier()

    pltpu.sync_copy(acce_sh.at[pl.ds(sid * _RPT, _RPT)],
                    oute_hbm.at[cid, pl.ds(sid * _RPT, _RPT)])


_sc_mesh = plsc.VectorSubcoreMesh(core_axis_name="c", subcore_axis_name="s")

_sc_aggr_x = pl.kernel(
    _sc_x_body,
    out_type=jax.ShapeDtypeStruct((_NC, _NPAD, _NV), jnp.float32),
    mesh=_sc_mesh,
    scratch_types=[
        pltpu.VMEM((_HALF, _CHUNK), jnp.int32),
        pltpu.VMEM((_HALF, _CHUNK), jnp.int32),
        pltpu.VMEM((_CHUNK, _NV), jnp.float32),
        pltpu.VMEM_SHARED((_NPAD, _NV), jnp.float32),
        pltpu.SemaphoreType.DMA,
    ],
)

_sc_aggr_e = pl.kernel(
    _sc_e_body,
    out_type=jax.ShapeDtypeStruct((_NC, _NPAD, _NE), jnp.float32),
    mesh=_sc_mesh,
    scratch_types=[
        pltpu.VMEM((_CPW, _CHUNK), jnp.int32),
        pltpu.VMEM((_CHUNK, _NE), jnp.float32),
        pltpu.VMEM_SHARED((_NPAD, _NE), jnp.float32),
    ],
)


def _tc_body(axp_ref, aep_ref, d_ref, wx_ref, we_ref, rp_ref, h_ref, ps_ref):
    i = pl.program_id(0)
    ax = axp_ref[0] + axp_ref[1]
    ae = aep_ref[0] + aep_ref[1]
    d = d_ref[...]
    f = 1.0 / d.astype(jnp.float32)
    sx = ax * f
    se = ae * f

    def body(b, acc):
        m = (d == b + 1).astype(jnp.float32)
        acc = acc + jnp.dot(sx * m, wx_ref[b], preferred_element_type=jnp.float32)
        acc = acc + jnp.dot(se * m, we_ref[b], preferred_element_type=jnp.float32)
        return acc

    acc = lax.fori_loop(0, _B, body, jnp.zeros((_R, _NV), jnp.float32))
    h = 1.0 / (1.0 + jnp.exp(-acc))
    h_ref[...] = h

    logits = jnp.dot(h, rp_ref[...], preferred_element_type=jnp.float32)
    lane = lax.broadcasted_iota(jnp.int32, (_R, _NV), 1)
    valid = lane < _NOUT
    mx = jnp.max(jnp.where(valid, logits, -1e30), axis=1, keepdims=True)
    e = jnp.where(valid, jnp.exp(logits - mx), 0.0)
    p = e / jnp.sum(e, axis=1, keepdims=True)
    colsum = jnp.sum(p, axis=0, keepdims=True)

    @pl.when(i == 0)
    def _():
        ps_ref[...] = jnp.zeros_like(ps_ref)

    ps_ref[...] += colsum


_tc_step = pl.pallas_call(
    _tc_body,
    grid=(_NBLK,),
    in_specs=[
        pl.BlockSpec((_NC, _R, _NV), lambda i: (0, i, 0)),
        pl.BlockSpec((_NC, _R, _NE), lambda i: (0, i, 0)),
        pl.BlockSpec((_R, 1), lambda i: (i, 0)),
        pl.BlockSpec((_B, _NV, _NV), lambda i: (0, 0, 0)),
        pl.BlockSpec((_B, _NE, _NV), lambda i: (0, 0, 0)),
        pl.BlockSpec((_NV, _NV), lambda i: (0, 0)),
    ],
    out_specs=[
        pl.BlockSpec((_R, _NV), lambda i: (i, 0)),
        pl.BlockSpec((1, _NV), lambda i: (0, 0)),
    ],
    out_shape=[
        jax.ShapeDtypeStruct((_N, _NV), jnp.float32),
        jax.ShapeDtypeStruct((1, _NV), jnp.float32),
    ],
)


def kernel(x, edge_index, edge_attr, node_degree, weights, readout_weights):
    src = edge_index[0].astype(jnp.int32)
    dst = edge_index[1].astype(jnp.int32)
    npadE = _EPAD - _E
    src2 = jnp.concatenate([src, jnp.zeros((npadE,), jnp.int32)]).reshape(_NCHUNKS, _CHUNK)
    dst2 = jnp.concatenate([dst, jnp.full((npadE,), _NPAD - 1, jnp.int32)]).reshape(_NCHUNKS, _CHUNK)
    ea = jnp.concatenate([edge_attr, jnp.zeros((npadE, _NE), jnp.float32)])
    d = jnp.clip(node_degree, _MIND, _MAXD).astype(jnp.int32).reshape(_N, 1)
    w_all = weights.reshape(_T, _B, _NV + _NE, _NV)
    wx = w_all[:, :, :_NV, :]
    we = w_all[:, :, _NV:, :]
    rp = jnp.pad(readout_weights.reshape(_T, _NV, _NOUT),
                 ((0, 0), (0, 0), (0, _NV - _NOUT)))

    h = x
    aep = _sc_aggr_e(ea, dst2)
    total = jnp.zeros((_NV,), jnp.float32)
    for t in range(_T):
        axp = _sc_aggr_x(h, src2, dst2)
        h, ps = _tc_step(axp, aep, d, wx[t], we[t], rp[t])
        total = total + ps[0]
    return total[:_NOUT]


# speedup vs baseline: 2.5276x; 1.0012x over previous
import functools

import jax
import jax.numpy as jnp
from jax import lax
from jax.experimental import pallas as pl
from jax.experimental.pallas import tpu as pltpu
from jax.experimental.pallas import tpu_sc as plsc

_N = 10000
_E = 320000
_NV = 128
_NE = 16
_MAXD = 32
_MIND = 1
_T = 4
_NOUT = 10
_B = _MAXD - _MIND + 1

_NC = 2
_NS = 16
_NW = _NC * _NS
_CHUNK = 128
_CPW = 80
_HALF = _CPW // 2
_EPAD = _NW * _CPW * _CHUNK
_NCHUNKS = _EPAD // _CHUNK
_NPAD = 10112
_RPT = _NPAD // _NS

_R = 1000
_NBLK = _N // _R


def _zero_acc(buf_v, acc_sh, sid, width):
    def zrow(r, carry):
        for c in range(width // 16):
            buf_v[r, pl.ds(c * 16, 16)] = jnp.zeros((16,), jnp.float32)
        return carry

    lax.fori_loop(0, _CHUNK, zrow, 0)
    for k in range(0, _RPT, _CHUNK):
        rows = min(_CHUNK, _RPT - k)
        pltpu.sync_copy(buf_v.at[pl.ds(0, rows)],
                        acc_sh.at[pl.ds(sid * _RPT + k, rows)])


def _sc_x_body(h_hbm, src_hbm, dst_hbm, outx_hbm,
               src_v, dst_v, rows_v, accx_sh, sem):
    cid = lax.axis_index("c")
    sid = lax.axis_index("s")
    w = sid * _NC + cid

    _zero_acc(rows_v, accx_sh, sid, _NV)
    plsc.subcore_barrier()

    def step(j, carry):
        pltpu.async_copy(h_hbm.at[src_v.at[j]], rows_v, sem).wait()
        pltpu.sync_copy(rows_v, accx_sh.at[dst_v.at[j]], add=True)
        return carry

    for hh in range(2):
        half_base = w * _CPW + hh * _HALF
        pltpu.sync_copy(src_hbm.at[pl.ds(half_base, _HALF)], src_v)
        pltpu.sync_copy(dst_hbm.at[pl.ds(half_base, _HALF)], dst_v)
        lax.fori_loop(0, _HALF, step, 0)
    plsc.subcore_barrier()

    pltpu.sync_copy(accx_sh.at[pl.ds(sid * _RPT, _RPT)],
                    outx_hbm.at[cid, pl.ds(sid * _RPT, _RPT)])


def _sc_e_body(ea_hbm, dst_hbm, oute_hbm, dst_v, erow_v, acce_sh):
    cid = lax.axis_index("c")
    sid = lax.axis_index("s")
    w = sid * _NC + cid

    _zero_acc(erow_v, acce_sh, sid, _NE)
    plsc.subcore_barrier()

    def step(j, carry):
        base = (w * _CPW + j) * _CHUNK
        pltpu.sync_copy(ea_hbm.at[pl.ds(base, _CHUNK)], erow_v)
        pltpu.sync_copy(erow_v, acce_sh.at[dst_v.at[j]], add=True)
        return carry

    pltpu.sync_copy(dst_hbm.at[pl.ds(w * _CPW, _CPW)], dst_v)
    lax.fori_loop(0, _CPW, step, 0)
    plsc.subcore_barrier()

    pltpu.sync_copy(acce_sh.at[pl.ds(sid * _RPT, _RPT)],
                    oute_hbm.at[cid, pl.ds(sid * _RPT, _RPT)])


_sc_mesh = plsc.VectorSubcoreMesh(core_axis_name="c", subcore_axis_name="s")

_sc_aggr_x = pl.kernel(
    _sc_x_body,
    out_type=jax.ShapeDtypeStruct((_NC, _NPAD, _NV), jnp.float32),
    mesh=_sc_mesh,
    scratch_types=[
        pltpu.VMEM((_HALF, _CHUNK), jnp.int32),
        pltpu.VMEM((_HALF, _CHUNK), jnp.int32),
        pltpu.VMEM((_CHUNK, _NV), jnp.float32),
        pltpu.VMEM_SHARED((_NPAD, _NV), jnp.float32),
        pltpu.SemaphoreType.DMA,
    ],
)

_sc_aggr_e = pl.kernel(
    _sc_e_body,
    out_type=jax.ShapeDtypeStruct((_NC, _NPAD, _NE), jnp.float32),
    mesh=_sc_mesh,
    scratch_types=[
        pltpu.VMEM((_CPW, _CHUNK), jnp.int32),
        pltpu.VMEM((_CHUNK, _NE), jnp.float32),
        pltpu.VMEM_SHARED((_NPAD, _NE), jnp.float32),
    ],
)


def _tc_body(axp_ref, aep_ref, d_ref, wx_ref, we_ref, rp_ref, h_ref, ps_ref):
    i = pl.program_id(0)
    ax = axp_ref[0] + axp_ref[1]
    ae = aep_ref[0] + aep_ref[1]
    d = d_ref[...]
    f = 1.0 / d.astype(jnp.float32)
    sx = ax * f
    se = ae * f

    def body(b, acc):
        m = (d == b + 1).astype(jnp.float32)
        acc = acc + jnp.dot(sx * m, wx_ref[b], preferred_element_type=jnp.float32)
        acc = acc + jnp.dot(se * m, we_ref[b], preferred_element_type=jnp.float32)
        return acc

    acc = lax.fori_loop(0, _B, body, jnp.zeros((_R, _NV), jnp.float32))
    h = 1.0 / (1.0 + jnp.exp(-acc))
    h_ref[...] = h

    logits = jnp.dot(h, rp_ref[...], preferred_element_type=jnp.float32)
    lane = lax.broadcasted_iota(jnp.int32, (_R, _NV), 1)
    valid = lane < _NOUT
    mx = jnp.max(jnp.where(valid, logits, -1e30), axis=1, keepdims=True)
    e = jnp.where(valid, jnp.exp(logits - mx), 0.0)
    p = e / jnp.sum(e, axis=1, keepdims=True)
    colsum = jnp.sum(p, axis=0, keepdims=True)

    @pl.when(i == 0)
    def _():
        ps_ref[...] = jnp.zeros_like(ps_ref)

    ps_ref[...] += colsum


_tc_step = pl.pallas_call(
    _tc_body,
    grid=(_NBLK,),
    in_specs=[
        pl.BlockSpec((_NC, _R, _NV), lambda i: (0, i, 0)),
        pl.BlockSpec((_NC, _R, _NE), lambda i: (0, i, 0)),
        pl.BlockSpec((_R, 1), lambda i: (i, 0)),
        pl.BlockSpec((_B, _NV, _NV), lambda i: (0, 0, 0)),
        pl.BlockSpec((_B, _NE, _NV), lambda i: (0, 0, 0)),
        pl.BlockSpec((_NV, _NV), lambda i: (0, 0)),
    ],
    out_specs=[
        pl.BlockSpec((_R, _NV), lambda i: (i, 0)),
        pl.BlockSpec((1, _NV), lambda i: (0, 0)),
    ],
    out_shape=[
        jax.ShapeDtypeStruct((_N, _NV), jnp.float32),
        jax.ShapeDtypeStruct((1, _NV), jnp.float32),
    ],
)


def kernel(x, edge_index, edge_attr, node_degree, weights, readout_weights):
    src = edge_index[0].astype(jnp.int32)
    dst = edge_index[1].astype(jnp.int32)
    npadE = _EPAD - _E
    src2 = jnp.concatenate([src, jnp.zeros((npadE,), jnp.int32)]).reshape(_NCHUNKS, _CHUNK)
    pad_dst = _N + (jnp.arange(npadE, dtype=jnp.int32) % (_NPAD - _N))
    dst2 = jnp.concatenate([dst, pad_dst]).reshape(_NCHUNKS, _CHUNK)
    ea = jnp.concatenate([edge_attr, jnp.zeros((npadE, _NE), jnp.float32)])
    d = jnp.clip(node_degree, _MIND, _MAXD).astype(jnp.int32).reshape(_N, 1)
    w_all = weights.reshape(_T, _B, _NV + _NE, _NV)
    wx = w_all[:, :, :_NV, :]
    we = w_all[:, :, _NV:, :]
    rp = jnp.pad(readout_weights.reshape(_T, _NV, _NOUT),
                 ((0, 0), (0, 0), (0, _NV - _NOUT)))

    h = x
    aep = _sc_aggr_e(ea, dst2)
    total = jnp.zeros((_NV,), jnp.float32)
    for t in range(_T):
        axp = _sc_aggr_x(h, src2, dst2)
        h, ps = _tc_step(axp, aep, d, wx[t], we[t], rp[t])
        total = total + ps[0]
    return total[:_NOUT]


# speedup vs baseline: 5.7295x; 2.2668x over previous
import functools

import jax
import jax.numpy as jnp
from jax import lax
from jax.experimental import pallas as pl
from jax.experimental.pallas import tpu as pltpu
from jax.experimental.pallas import tpu_sc as plsc

_N = 10000
_E = 320000
_NV = 128
_NE = 16
_MAXD = 32
_MIND = 1
_T = 4
_NOUT = 10
_B = _MAXD - _MIND + 1

_NC = 2
_NS = 16
_NW = _NC * _NS
_CHUNK = 80
_CPW = 125
_CPWPAD = 128
_STAGES = ((0, 64, 64), (64, 64, 61))
_NCHUNKS = _NW * _CPWPAD
_NPAD = 10112
_RPT = _NPAD // _NS

_R = 1000
_NBLK = _N // _R


def _zero_acc(buf_v, acc_sh, sid, width):
    def zrow(r, carry):
        for c in range(width // 16):
            buf_v[r, pl.ds(c * 16, 16)] = jnp.zeros((16,), jnp.float32)
        return carry

    lax.fori_loop(0, _CHUNK, zrow, 0)
    for k in range(0, _RPT, _CHUNK):
        rows = min(_CHUNK, _RPT - k)
        pltpu.sync_copy(buf_v.at[pl.ds(0, rows)],
                        acc_sh.at[pl.ds(sid * _RPT + k, rows)])


def _sc_x_body(h_hbm, src_hbm, dst_hbm, outx_hbm,
               src_v, dst_v, rows_a, rows_b, accx_sh, sem_a, sem_b):
    cid = lax.axis_index("c")
    sid = lax.axis_index("s")
    w = sid * _NC + cid

    _zero_acc(rows_a, accx_sh, sid, _NV)
    plsc.subcore_barrier()

    def wait_rows(buf, sem):
        pltpu.make_async_copy(h_hbm.at[pl.ds(0, _CHUNK)], buf, sem).wait()

    def pipe(nloc, j, carry):
        wait_rows(rows_a, sem_a)
        pltpu.sync_copy(rows_a, accx_sh.at[dst_v.at[2 * j]], add=True)

        @pl.when(2 * j + 2 < nloc)
        def _():
            pltpu.async_copy(h_hbm.at[src_v.at[2 * j + 2]], rows_a, sem_a)

        wait_rows(rows_b, sem_b)
        pltpu.sync_copy(rows_b, accx_sh.at[dst_v.at[2 * j + 1]], add=True)

        @pl.when(2 * j + 3 < nloc)
        def _():
            pltpu.async_copy(h_hbm.at[src_v.at[2 * j + 3]], rows_b, sem_b)

        return carry

    for off, nstage, nproc in _STAGES:
        base = w * _CPWPAD + off
        npipe = nproc - (nproc % 2)
        pltpu.sync_copy(src_hbm.at[pl.ds(base, nstage)], src_v)
        pltpu.sync_copy(dst_hbm.at[pl.ds(base, nstage)], dst_v)
        pltpu.async_copy(h_hbm.at[src_v.at[0]], rows_a, sem_a)
        pltpu.async_copy(h_hbm.at[src_v.at[1]], rows_b, sem_b)
        lax.fori_loop(0, npipe // 2, functools.partial(pipe, npipe), 0)
        for tail in range(npipe, nproc):
            pltpu.async_copy(h_hbm.at[src_v.at[tail]], rows_a, sem_a).wait()
            pltpu.sync_copy(rows_a, accx_sh.at[dst_v.at[tail]], add=True)
    plsc.subcore_barrier()

    pltpu.sync_copy(accx_sh.at[pl.ds(sid * _RPT, _RPT)],
                    outx_hbm.at[cid, pl.ds(sid * _RPT, _RPT)])


def _sc_e_body(ea_hbm, dst_hbm, oute_hbm, dst_v, erow_v, acce_sh):
    cid = lax.axis_index("c")
    sid = lax.axis_index("s")
    w = sid * _NC + cid

    _zero_acc(erow_v, acce_sh, sid, _NE)
    plsc.subcore_barrier()

    def step(j, carry):
        base = (w * _CPW + j) * _CHUNK
        pltpu.sync_copy(ea_hbm.at[pl.ds(base, _CHUNK)], erow_v)
        pltpu.sync_copy(erow_v, acce_sh.at[dst_v.at[j]], add=True)
        return carry

    pltpu.sync_copy(dst_hbm.at[pl.ds(w * _CPWPAD, _CPWPAD)], dst_v)
    lax.fori_loop(0, _CPW, step, 0)
    plsc.subcore_barrier()

    pltpu.sync_copy(acce_sh.at[pl.ds(sid * _RPT, _RPT)],
                    oute_hbm.at[cid, pl.ds(sid * _RPT, _RPT)])


_sc_mesh = plsc.VectorSubcoreMesh(core_axis_name="c", subcore_axis_name="s")

_sc_aggr_x = pl.kernel(
    _sc_x_body,
    out_type=jax.ShapeDtypeStruct((_NC, _NPAD, _NV), jnp.float32),
    mesh=_sc_mesh,
    scratch_types=[
        pltpu.VMEM((_STAGES[0][1], _CHUNK), jnp.int32),
        pltpu.VMEM((_STAGES[0][1], _CHUNK), jnp.int32),
        pltpu.VMEM((_CHUNK, _NV), jnp.float32),
        pltpu.VMEM((_CHUNK, _NV), jnp.float32),
        pltpu.VMEM_SHARED((_NPAD, _NV), jnp.float32),
        pltpu.SemaphoreType.DMA,
        pltpu.SemaphoreType.DMA,
    ],
)

_sc_aggr_e = pl.kernel(
    _sc_e_body,
    out_type=jax.ShapeDtypeStruct((_NC, _NPAD, _NE), jnp.float32),
    mesh=_sc_mesh,
    scratch_types=[
        pltpu.VMEM((_CPWPAD, _CHUNK), jnp.int32),
        pltpu.VMEM((_CHUNK, _NE), jnp.float32),
        pltpu.VMEM_SHARED((_NPAD, _NE), jnp.float32),
    ],
)


def _tc_body(axp_ref, aep_ref, d_ref, wx_ref, we_ref, rp_ref, h_ref, ps_ref):
    i = pl.program_id(0)
    ax = axp_ref[0] + axp_ref[1]
    ae = aep_ref[0] + aep_ref[1]
    d = d_ref[...]
    f = 1.0 / d.astype(jnp.float32)
    sx = ax * f
    se = ae * f

    def body(b, acc):
        m = (d == b + 1).astype(jnp.float32)
        acc = acc + jnp.dot(sx * m, wx_ref[b], preferred_element_type=jnp.float32)
        acc = acc + jnp.dot(se * m, we_ref[b], preferred_element_type=jnp.float32)
        return acc

    acc = lax.fori_loop(0, _B, body, jnp.zeros((_R, _NV), jnp.float32))
    h = 1.0 / (1.0 + jnp.exp(-acc))
    h_ref[...] = h

    logits = jnp.dot(h, rp_ref[...], preferred_element_type=jnp.float32)
    lane = lax.broadcasted_iota(jnp.int32, (_R, _NV), 1)
    valid = lane < _NOUT
    mx = jnp.max(jnp.where(valid, logits, -1e30), axis=1, keepdims=True)
    e = jnp.where(valid, jnp.exp(logits - mx), 0.0)
    p = e / jnp.sum(e, axis=1, keepdims=True)
    colsum = jnp.sum(p, axis=0, keepdims=True)

    @pl.when(i == 0)
    def _():
        ps_ref[...] = jnp.zeros_like(ps_ref)

    ps_ref[...] += colsum


_tc_step = pl.pallas_call(
    _tc_body,
    grid=(_NBLK,),
    in_specs=[
        pl.BlockSpec((_NC, _R, _NV), lambda i: (0, i, 0)),
        pl.BlockSpec((_NC, _R, _NE), lambda i: (0, i, 0)),
        pl.BlockSpec((_R, 1), lambda i: (i, 0)),
        pl.BlockSpec((_B, _NV, _NV), lambda i: (0, 0, 0)),
        pl.BlockSpec((_B, _NE, _NV), lambda i: (0, 0, 0)),
        pl.BlockSpec((_NV, _NV), lambda i: (0, 0)),
    ],
    out_specs=[
        pl.BlockSpec((_R, _NV), lambda i: (i, 0)),
        pl.BlockSpec((1, _NV), lambda i: (0, 0)),
    ],
    out_shape=[
        jax.ShapeDtypeStruct((_N, _NV), jnp.float32),
        jax.ShapeDtypeStruct((1, _NV), jnp.float32),
    ],
)


def kernel(x, edge_index, edge_attr, node_degree, weights, readout_weights):
    def pad_idx(v):
        v3 = v.astype(jnp.int32).reshape(_NW, _CPW, _CHUNK)
        v3 = jnp.pad(v3, ((0, 0), (0, _CPWPAD - _CPW), (0, 0)))
        return v3.reshape(_NCHUNKS, _CHUNK)

    src2 = pad_idx(edge_index[0])
    dst2 = pad_idx(edge_index[1])
    ea = edge_attr
    d = jnp.clip(node_degree, _MIND, _MAXD).astype(jnp.int32).reshape(_N, 1)
    w_all = weights.reshape(_T, _B, _NV + _NE, _NV)
    wx = w_all[:, :, :_NV, :]
    we = w_all[:, :, _NV:, :]
    rp = jnp.pad(readout_weights.reshape(_T, _NV, _NOUT),
                 ((0, 0), (0, 0), (0, _NV - _NOUT)))

    h = x
    aep = _sc_aggr_e(ea, dst2)
    total = jnp.zeros((_NV,), jnp.float32)
    for t in range(_T):
        axp = _sc_aggr_x(h, src2, dst2)
        h, ps = _tc_step(axp, aep, d, wx[t], we[t], rp[t])
        total = total + ps[0]
    return total[:_NOUT]


# speedup vs baseline: 5.9858x; 1.0447x over previous
import functools

import jax
import jax.numpy as jnp
from jax import lax
from jax.experimental import pallas as pl
from jax.experimental.pallas import tpu as pltpu
from jax.experimental.pallas import tpu_sc as plsc

_N = 10000
_E = 320000
_NV = 128
_NE = 16
_MAXD = 32
_MIND = 1
_T = 4
_NOUT = 10
_B = _MAXD - _MIND + 1

_NC = 2
_NS = 16
_NW = _NC * _NS
_CHUNK = 80
_CPW = 125
_CPWPAD = 128
_STAGES = ((0, 64, 64), (64, 64, 61))
_NCHUNKS = _NW * _CPWPAD
_NPAD = 10112
_RPT = _NPAD // _NS

_R = 1000
_NBLK = _N // _R


def _zero_acc(buf_v, acc_sh, sid, width):
    def zrow(r, carry):
        for c in range(width // 16):
            buf_v[r, pl.ds(c * 16, 16)] = jnp.zeros((16,), jnp.float32)
        return carry

    lax.fori_loop(0, _CHUNK, zrow, 0)
    for k in range(0, _RPT, _CHUNK):
        rows = min(_CHUNK, _RPT - k)
        pltpu.sync_copy(buf_v.at[pl.ds(0, rows)],
                        acc_sh.at[pl.ds(sid * _RPT + k, rows)])


def _sc_x_body(h_hbm, src_hbm, dst_hbm, outx_hbm,
               src_v, dst_v, rows_a, rows_b, accx_sh, sem_a, sem_b):
    cid = lax.axis_index("c")
    sid = lax.axis_index("s")
    w = sid * _NC + cid

    _zero_acc(rows_a, accx_sh, sid, _NV)
    plsc.subcore_barrier()

    def wait_rows(buf, sem):
        pltpu.make_async_copy(h_hbm.at[pl.ds(0, _CHUNK)], buf, sem).wait()

    def pipe(nloc, j, carry):
        wait_rows(rows_a, sem_a)
        pltpu.sync_copy(rows_a, accx_sh.at[dst_v.at[2 * j]], add=True)

        @pl.when(2 * j + 2 < nloc)
        def _():
            pltpu.async_copy(h_hbm.at[src_v.at[2 * j + 2]], rows_a, sem_a)

        wait_rows(rows_b, sem_b)
        pltpu.sync_copy(rows_b, accx_sh.at[dst_v.at[2 * j + 1]], add=True)

        @pl.when(2 * j + 3 < nloc)
        def _():
            pltpu.async_copy(h_hbm.at[src_v.at[2 * j + 3]], rows_b, sem_b)

        return carry

    for off, nstage, nproc in _STAGES:
        base = w * _CPWPAD + off
        npipe = nproc - (nproc % 2)
        pltpu.sync_copy(src_hbm.at[pl.ds(base, nstage)], src_v)
        pltpu.sync_copy(dst_hbm.at[pl.ds(base, nstage)], dst_v)
        pltpu.async_copy(h_hbm.at[src_v.at[0]], rows_a, sem_a)
        pltpu.async_copy(h_hbm.at[src_v.at[1]], rows_b, sem_b)
        lax.fori_loop(0, npipe // 2, functools.partial(pipe, npipe), 0)
        for tail in range(npipe, nproc):
            pltpu.async_copy(h_hbm.at[src_v.at[tail]], rows_a, sem_a).wait()
            pltpu.sync_copy(rows_a, accx_sh.at[dst_v.at[tail]], add=True)
    plsc.subcore_barrier()

    pltpu.sync_copy(accx_sh.at[pl.ds(sid * _RPT, _RPT)],
                    outx_hbm.at[cid, pl.ds(sid * _RPT, _RPT)])


def _sc_e_body(ea_hbm, dst_hbm, oute_hbm, dst_v, erow_a, erow_b, acce_sh,
               sem_a, sem_b):
    cid = lax.axis_index("c")
    sid = lax.axis_index("s")
    w = sid * _NC + cid

    _zero_acc(erow_a, acce_sh, sid, _NE)
    plsc.subcore_barrier()

    def wait_rows(buf, sem):
        pltpu.make_async_copy(ea_hbm.at[pl.ds(0, _CHUNK)], buf, sem).wait()

    base0 = w * _CPW * _CHUNK

    def pipe(j, carry):
        wait_rows(erow_a, sem_a)
        pltpu.sync_copy(erow_a, acce_sh.at[dst_v.at[2 * j]], add=True)

        @pl.when(2 * j + 2 < _CPW - 1)
        def _():
            pltpu.async_copy(ea_hbm.at[pl.ds(base0 + (2 * j + 2) * _CHUNK, _CHUNK)],
                             erow_a, sem_a)

        wait_rows(erow_b, sem_b)
        pltpu.sync_copy(erow_b, acce_sh.at[dst_v.at[2 * j + 1]], add=True)

        @pl.when(2 * j + 3 < _CPW)
        def _():
            pltpu.async_copy(ea_hbm.at[pl.ds(base0 + (2 * j + 3) * _CHUNK, _CHUNK)],
                             erow_b, sem_b)

        return carry

    pltpu.sync_copy(dst_hbm.at[pl.ds(w * _CPWPAD, _CPWPAD)], dst_v)
    pltpu.async_copy(ea_hbm.at[pl.ds(base0, _CHUNK)], erow_a, sem_a)
    pltpu.async_copy(ea_hbm.at[pl.ds(base0 + _CHUNK, _CHUNK)], erow_b, sem_b)
    lax.fori_loop(0, (_CPW - 1) // 2, pipe, 0)
    pltpu.async_copy(ea_hbm.at[pl.ds(base0 + (_CPW - 1) * _CHUNK, _CHUNK)],
                     erow_a, sem_a).wait()
    pltpu.sync_copy(erow_a, acce_sh.at[dst_v.at[_CPW - 1]], add=True)
    plsc.subcore_barrier()

    pltpu.sync_copy(acce_sh.at[pl.ds(sid * _RPT, _RPT)],
                    oute_hbm.at[cid, pl.ds(sid * _RPT, _RPT)])


_sc_mesh = plsc.VectorSubcoreMesh(core_axis_name="c", subcore_axis_name="s")

_sc_aggr_x = pl.kernel(
    _sc_x_body,
    out_type=jax.ShapeDtypeStruct((_NC, _NPAD, _NV), jnp.float32),
    mesh=_sc_mesh,
    scratch_types=[
        pltpu.VMEM((_STAGES[0][1], _CHUNK), jnp.int32),
        pltpu.VMEM((_STAGES[0][1], _CHUNK), jnp.int32),
        pltpu.VMEM((_CHUNK, _NV), jnp.float32),
        pltpu.VMEM((_CHUNK, _NV), jnp.float32),
        pltpu.VMEM_SHARED((_NPAD, _NV), jnp.float32),
        pltpu.SemaphoreType.DMA,
        pltpu.SemaphoreType.DMA,
    ],
)

_sc_aggr_e = pl.kernel(
    _sc_e_body,
    out_type=jax.ShapeDtypeStruct((_NC, _NPAD, _NE), jnp.float32),
    mesh=_sc_mesh,
    scratch_types=[
        pltpu.VMEM((_CPWPAD, _CHUNK), jnp.int32),
        pltpu.VMEM((_CHUNK, _NE), jnp.float32),
        pltpu.VMEM((_CHUNK, _NE), jnp.float32),
        pltpu.VMEM_SHARED((_NPAD, _NE), jnp.float32),
        pltpu.SemaphoreType.DMA,
        pltpu.SemaphoreType.DMA,
    ],
)


def _tc_body(axp_ref, aep_ref, d_ref, w_ref, rp_ref, h_ref, ps_ref):
    i = pl.program_id(0)
    ax = axp_ref[0] + axp_ref[1]
    ae = aep_ref[0] + aep_ref[1]
    d = d_ref[...]
    f = 1.0 / d.astype(jnp.float32)
    sx = ax * f
    se = ae * f

    def body(b, acc):
        m = (d == b + 1).astype(jnp.float32)
        acc = acc + jnp.dot(sx * m, w_ref[b, : _NV, :],
                            preferred_element_type=jnp.float32)
        acc = acc + jnp.dot(se * m, w_ref[b, _NV :, :],
                            preferred_element_type=jnp.float32)
        return acc

    acc = lax.fori_loop(0, _B, body, jnp.zeros((_R, _NV), jnp.float32))
    h = 1.0 / (1.0 + jnp.exp(-acc))
    h_ref[...] = h

    logits = jnp.dot(h, rp_ref[...], preferred_element_type=jnp.float32)
    lane = lax.broadcasted_iota(jnp.int32, (_R, _NE), 1)
    valid = lane < _NOUT
    mx = jnp.max(jnp.where(valid, logits, -1e30), axis=1, keepdims=True)
    e = jnp.where(valid, jnp.exp(logits - mx), 0.0)
    p = e / jnp.sum(e, axis=1, keepdims=True)
    colsum = jnp.sum(p, axis=0, keepdims=True)

    @pl.when(i == 0)
    def _():
        ps_ref[...] = jnp.zeros_like(ps_ref)

    ps_ref[...] += colsum


_tc_step = pl.pallas_call(
    _tc_body,
    grid=(_NBLK,),
    in_specs=[
        pl.BlockSpec((_NC, _R, _NV), lambda i: (0, i, 0)),
        pl.BlockSpec((_NC, _R, _NE), lambda i: (0, i, 0)),
        pl.BlockSpec((_R, 1), lambda i: (i, 0)),
        pl.BlockSpec((_B, _NV + _NE, _NV), lambda i: (0, 0, 0)),
        pl.BlockSpec((_NV, _NE), lambda i: (0, 0)),
    ],
    out_specs=[
        pl.BlockSpec((_R, _NV), lambda i: (i, 0)),
        pl.BlockSpec((1, _NE), lambda i: (0, 0)),
    ],
    out_shape=[
        jax.ShapeDtypeStruct((_N, _NV), jnp.float32),
        jax.ShapeDtypeStruct((1, _NE), jnp.float32),
    ],
)


def kernel(x, edge_index, edge_attr, node_degree, weights, readout_weights):
    def pad_idx(v):
        v3 = v.astype(jnp.int32).reshape(_NW, _CPW, _CHUNK)
        v3 = jnp.pad(v3, ((0, 0), (0, _CPWPAD - _CPW), (0, 0)))
        return v3.reshape(_NCHUNKS, _CHUNK)

    src2 = pad_idx(edge_index[0])
    dst2 = pad_idx(edge_index[1])
    ea = edge_attr
    d = jnp.clip(node_degree, _MIND, _MAXD).astype(jnp.int32).reshape(_N, 1)
    w_all = weights.reshape(_T, _B, _NV + _NE, _NV)
    rp = jnp.pad(readout_weights.reshape(_T, _NV, _NOUT),
                 ((0, 0), (0, 0), (0, _NE - _NOUT)))

    h = x
    aep = _sc_aggr_e(ea, dst2)
    total = jnp.zeros((_NE,), jnp.float32)
    for t in range(_T):
        axp = _sc_aggr_x(h, src2, dst2)
        h, ps = _tc_step(axp, aep, d, w_all[t], rp[t])
        total = total + ps[0]
    return total[:_NOUT]


# speedup vs baseline: 6.8385x; 1.1425x over previous
import functools

import jax
import jax.numpy as jnp
from jax import lax
from jax.experimental import pallas as pl
from jax.experimental.pallas import tpu as pltpu
from jax.experimental.pallas import tpu_sc as plsc

_N = 10000
_E = 320000
_NV = 128
_NE = 16
_MAXD = 32
_MIND = 1
_T = 4
_NOUT = 10
_B = _MAXD - _MIND + 1

_NC = 2
_NS = 16
_NW = _NC * _NS
_CHUNK = 80
_CPW = 125
_CPWPAD = 128
_STAGES = ((0, 64, 64), (64, 64, 61))
_NCHUNKS = _NW * _CPWPAD
_NPAD = 10112
_RPT = _NPAD // _NS

_R = 1000
_NBLK = _N // _R


def _zero_acc(buf_v, acc_sh, sid, width):
    def zrow(r, carry):
        for c in range(width // 16):
            buf_v[r, pl.ds(c * 16, 16)] = jnp.zeros((16,), jnp.float32)
        return carry

    lax.fori_loop(0, _CHUNK, zrow, 0)
    for k in range(0, _RPT, _CHUNK):
        rows = min(_CHUNK, _RPT - k)
        pltpu.sync_copy(buf_v.at[pl.ds(0, rows)],
                        acc_sh.at[pl.ds(sid * _RPT + k, rows)])


def _sc_x_body(h_hbm, src_hbm, dst_hbm, outx_hbm,
               src_v, dst_v, rows_a, rows_b, accx_sh, sem_a, sem_b):
    cid = lax.axis_index("c")
    sid = lax.axis_index("s")
    w = sid * _NC + cid

    _zero_acc(rows_a, accx_sh, sid, _NV)
    plsc.subcore_barrier()

    def wait_rows(buf, sem):
        pltpu.make_async_copy(h_hbm.at[pl.ds(0, _CHUNK)], buf, sem).wait()

    def pipe(nloc, j, carry):
        wait_rows(rows_a, sem_a)
        pltpu.sync_copy(rows_a, accx_sh.at[dst_v.at[2 * j]], add=True)

        @pl.when(2 * j + 2 < nloc)
        def _():
            pltpu.async_copy(h_hbm.at[src_v.at[2 * j + 2]], rows_a, sem_a)

        wait_rows(rows_b, sem_b)
        pltpu.sync_copy(rows_b, accx_sh.at[dst_v.at[2 * j + 1]], add=True)

        @pl.when(2 * j + 3 < nloc)
        def _():
            pltpu.async_copy(h_hbm.at[src_v.at[2 * j + 3]], rows_b, sem_b)

        return carry

    for off, nstage, nproc in _STAGES:
        base = w * _CPWPAD + off
        npipe = nproc - (nproc % 2)
        pltpu.sync_copy(src_hbm.at[pl.ds(base, nstage)], src_v)
        pltpu.sync_copy(dst_hbm.at[pl.ds(base, nstage)], dst_v)
        pltpu.async_copy(h_hbm.at[src_v.at[0]], rows_a, sem_a)
        pltpu.async_copy(h_hbm.at[src_v.at[1]], rows_b, sem_b)
        lax.fori_loop(0, npipe // 2, functools.partial(pipe, npipe), 0)
        for tail in range(npipe, nproc):
            pltpu.async_copy(h_hbm.at[src_v.at[tail]], rows_a, sem_a).wait()
            pltpu.sync_copy(rows_a, accx_sh.at[dst_v.at[tail]], add=True)
    plsc.subcore_barrier()

    pltpu.sync_copy(accx_sh.at[pl.ds(sid * _RPT, _RPT)],
                    outx_hbm.at[cid, pl.ds(sid * _RPT, _RPT)])


def _sc_e_body(ea_hbm, dst_hbm, oute_hbm, dst_v, erow_a, erow_b, acce_sh,
               sem_a, sem_b):
    cid = lax.axis_index("c")
    sid = lax.axis_index("s")
    w = sid * _NC + cid

    _zero_acc(erow_a, acce_sh, sid, _NE)
    plsc.subcore_barrier()

    def wait_rows(buf, sem):
        pltpu.make_async_copy(ea_hbm.at[pl.ds(0, _CHUNK)], buf, sem).wait()

    base0 = w * _CPW * _CHUNK

    def pipe(j, carry):
        wait_rows(erow_a, sem_a)
        pltpu.sync_copy(erow_a, acce_sh.at[dst_v.at[2 * j]], add=True)

        @pl.when(2 * j + 2 < _CPW - 1)
        def _():
            pltpu.async_copy(ea_hbm.at[pl.ds(base0 + (2 * j + 2) * _CHUNK, _CHUNK)],
                             erow_a, sem_a)

        wait_rows(erow_b, sem_b)
        pltpu.sync_copy(erow_b, acce_sh.at[dst_v.at[2 * j + 1]], add=True)

        @pl.when(2 * j + 3 < _CPW)
        def _():
            pltpu.async_copy(ea_hbm.at[pl.ds(base0 + (2 * j + 3) * _CHUNK, _CHUNK)],
                             erow_b, sem_b)

        return carry

    pltpu.sync_copy(dst_hbm.at[pl.ds(w * _CPWPAD, _CPWPAD)], dst_v)
    pltpu.async_copy(ea_hbm.at[pl.ds(base0, _CHUNK)], erow_a, sem_a)
    pltpu.async_copy(ea_hbm.at[pl.ds(base0 + _CHUNK, _CHUNK)], erow_b, sem_b)
    lax.fori_loop(0, (_CPW - 1) // 2, pipe, 0)
    pltpu.async_copy(ea_hbm.at[pl.ds(base0 + (_CPW - 1) * _CHUNK, _CHUNK)],
                     erow_a, sem_a).wait()
    pltpu.sync_copy(erow_a, acce_sh.at[dst_v.at[_CPW - 1]], add=True)
    plsc.subcore_barrier()

    pltpu.sync_copy(acce_sh.at[pl.ds(sid * _RPT, _RPT)],
                    oute_hbm.at[cid, pl.ds(sid * _RPT, _RPT)])


_sc_mesh = plsc.VectorSubcoreMesh(core_axis_name="c", subcore_axis_name="s")

_sc_aggr_x = pl.kernel(
    _sc_x_body,
    out_type=jax.ShapeDtypeStruct((_NC, _NPAD, _NV), jnp.float32),
    mesh=_sc_mesh,
    scratch_types=[
        pltpu.VMEM((_STAGES[0][1], _CHUNK), jnp.int32),
        pltpu.VMEM((_STAGES[0][1], _CHUNK), jnp.int32),
        pltpu.VMEM((_CHUNK, _NV), jnp.float32),
        pltpu.VMEM((_CHUNK, _NV), jnp.float32),
        pltpu.VMEM_SHARED((_NPAD, _NV), jnp.float32),
        pltpu.SemaphoreType.DMA,
        pltpu.SemaphoreType.DMA,
    ],
)

_sc_aggr_e = pl.kernel(
    _sc_e_body,
    out_type=jax.ShapeDtypeStruct((_NC, _NPAD, _NE), jnp.float32),
    mesh=_sc_mesh,
    scratch_types=[
        pltpu.VMEM((_CPWPAD, _CHUNK), jnp.int32),
        pltpu.VMEM((_CHUNK, _NE), jnp.float32),
        pltpu.VMEM((_CHUNK, _NE), jnp.float32),
        pltpu.VMEM_SHARED((_NPAD, _NE), jnp.float32),
        pltpu.SemaphoreType.DMA,
        pltpu.SemaphoreType.DMA,
    ],
)


def _tc_body(axp_ref, aep_ref, d_ref, w_ref, rp_ref, h_ref, ps_ref):
    i = pl.program_id(0)
    ax = axp_ref[0] + axp_ref[1]
    ae = aep_ref[0] + aep_ref[1]
    d = d_ref[...]
    f = 1.0 / d.astype(jnp.float32)
    sx = ax * f
    se = ae * f

    sxe = jnp.concatenate([sx, se], axis=1).astype(jnp.bfloat16)

    def body(b, acc):
        m = (d == b + 1).astype(jnp.bfloat16)
        acc = acc + jnp.dot(sxe * m, w_ref[b],
                            preferred_element_type=jnp.float32)
        return acc

    acc = lax.fori_loop(0, _B, body, jnp.zeros((_R, _NV), jnp.float32))
    h = 1.0 / (1.0 + jnp.exp(-acc))
    h_ref[...] = h

    logits = jnp.dot(h, rp_ref[...], preferred_element_type=jnp.float32)
    lane = lax.broadcasted_iota(jnp.int32, (_R, _NE), 1)
    valid = lane < _NOUT
    mx = jnp.max(jnp.where(valid, logits, -1e30), axis=1, keepdims=True)
    e = jnp.where(valid, jnp.exp(logits - mx), 0.0)
    p = e / jnp.sum(e, axis=1, keepdims=True)
    colsum = jnp.sum(p, axis=0, keepdims=True)

    @pl.when(i == 0)
    def _():
        ps_ref[...] = jnp.zeros_like(ps_ref)

    ps_ref[...] += colsum


_tc_step = pl.pallas_call(
    _tc_body,
    grid=(_NBLK,),
    in_specs=[
        pl.BlockSpec((_NC, _R, _NV), lambda i: (0, i, 0)),
        pl.BlockSpec((_NC, _R, _NE), lambda i: (0, i, 0)),
        pl.BlockSpec((_R, 1), lambda i: (i, 0)),
        pl.BlockSpec((_B, _NV + _NE, _NV), lambda i: (0, 0, 0)),
        pl.BlockSpec((_NV, _NE), lambda i: (0, 0)),
    ],
    out_specs=[
        pl.BlockSpec((_R, _NV), lambda i: (i, 0)),
        pl.BlockSpec((1, _NE), lambda i: (0, 0)),
    ],
    out_shape=[
        jax.ShapeDtypeStruct((_N, _NV), jnp.float32),
        jax.ShapeDtypeStruct((1, _NE), jnp.float32),
    ],
)


def kernel(x, edge_index, edge_attr, node_degree, weights, readout_weights):
    def pad_idx(v):
        v3 = v.astype(jnp.int32).reshape(_NW, _CPW, _CHUNK)
        v3 = jnp.pad(v3, ((0, 0), (0, _CPWPAD - _CPW), (0, 0)))
        return v3.reshape(_NCHUNKS, _CHUNK)

    src2 = pad_idx(edge_index[0])
    dst2 = pad_idx(edge_index[1])
    ea = edge_attr
    d = jnp.clip(node_degree, _MIND, _MAXD).astype(jnp.int32).reshape(_N, 1)
    w_all = weights.reshape(_T, _B, _NV + _NE, _NV).astype(jnp.bfloat16)
    rp = jnp.pad(readout_weights.reshape(_T, _NV, _NOUT),
                 ((0, 0), (0, 0), (0, _NE - _NOUT)))

    h = x
    aep = _sc_aggr_e(ea, dst2)
    total = jnp.zeros((_NE,), jnp.float32)
    for t in range(_T):
        axp = _sc_aggr_x(h, src2, dst2)
        h, ps = _tc_step(axp, aep, d, w_all[t], rp[t])
        total = total + ps[0]
    return total[:_NOUT]


# speedup vs baseline: 7.5571x; 1.1051x over previous
import functools

import jax
import jax.numpy as jnp
from jax import lax
from jax.experimental import pallas as pl
from jax.experimental.pallas import tpu as pltpu
from jax.experimental.pallas import tpu_sc as plsc

_N = 10000
_E = 320000
_NV = 128
_NE = 16
_MAXD = 32
_MIND = 1
_T = 4
_NOUT = 10
_B = _MAXD - _MIND + 1

_NC = 2
_NS = 16
_NW = _NC * _NS
_CHUNK = 80
_CPW = 125
_CPWPAD = 128
_STAGES = ((0, 64, 64), (64, 64, 61))
_NCHUNKS = _NW * _CPWPAD
_NPAD = 10112
_RPT = _NPAD // _NS

_R = 2000
_NBLK = _N // _R


_DEPTH = 2


def _zero_acc(buf_v, acc_sh, sid):
    width = buf_v.shape[1]
    def zrow(r, carry):
        for c in range(width // 16):
            buf_v[r, pl.ds(c * 16, 16)] = jnp.zeros((16,), jnp.float32)
        return carry

    lax.fori_loop(0, _CHUNK, zrow, 0)
    for k in range(0, _RPT, _CHUNK):
        rows = min(_CHUNK, _RPT - k)
        pltpu.sync_copy(buf_v.at[pl.ds(0, rows)],
                        acc_sh.at[pl.ds(sid * _RPT + k, rows)])


def _sc_x_body(h_hbm, src_hbm, dst_hbm, outx_hbm,
               src_v, dst_v, rows_a, rows_b, accx_sh, sem_a, sem_b):
    cid = lax.axis_index("c")
    sid = lax.axis_index("s")
    w = sid * _NC + cid

    _zero_acc(rows_a, accx_sh, sid)
    plsc.subcore_barrier()

    def wait_rows(buf, sem):
        pltpu.make_async_copy(h_hbm.at[pl.ds(0, _CHUNK)], buf, sem).wait()

    def pipe(nloc, j, carry):
        wait_rows(rows_a, sem_a)
        pltpu.sync_copy(rows_a, accx_sh.at[dst_v.at[2 * j]], add=True)

        @pl.when(2 * j + 2 < nloc)
        def _():
            pltpu.async_copy(h_hbm.at[src_v.at[2 * j + 2]], rows_a, sem_a)

        wait_rows(rows_b, sem_b)
        pltpu.sync_copy(rows_b, accx_sh.at[dst_v.at[2 * j + 1]], add=True)

        @pl.when(2 * j + 3 < nloc)
        def _():
            pltpu.async_copy(h_hbm.at[src_v.at[2 * j + 3]], rows_b, sem_b)

        return carry

    for off, nstage, nproc in _STAGES:
        base = w * _CPWPAD + off
        npipe = nproc - (nproc % 2)
        pltpu.sync_copy(src_hbm.at[pl.ds(base, nstage)], src_v)
        pltpu.sync_copy(dst_hbm.at[pl.ds(base, nstage)], dst_v)
        pltpu.async_copy(h_hbm.at[src_v.at[0]], rows_a, sem_a)
        pltpu.async_copy(h_hbm.at[src_v.at[1]], rows_b, sem_b)
        lax.fori_loop(0, npipe // 2, functools.partial(pipe, npipe), 0)
        for tail in range(npipe, nproc):
            pltpu.async_copy(h_hbm.at[src_v.at[tail]], rows_a, sem_a).wait()
            pltpu.sync_copy(rows_a, accx_sh.at[dst_v.at[tail]], add=True)
    plsc.subcore_barrier()

    pltpu.sync_copy(accx_sh.at[pl.ds(sid * _RPT, _RPT)],
                    outx_hbm.at[cid, pl.ds(sid * _RPT, _RPT)])


def _sc_e_body(ea_hbm, dst_hbm, oute_hbm, dst_v, erow_a, erow_b, acce_sh,
               sem_a, sem_b):
    cid = lax.axis_index("c")
    sid = lax.axis_index("s")
    w = sid * _NC + cid

    _zero_acc(erow_a, acce_sh, sid)
    plsc.subcore_barrier()

    def wait_rows(buf, sem):
        pltpu.make_async_copy(ea_hbm.at[pl.ds(0, _CHUNK)], buf, sem).wait()

    base0 = w * _CPW * _CHUNK

    def pipe(j, carry):
        wait_rows(erow_a, sem_a)
        pltpu.sync_copy(erow_a, acce_sh.at[dst_v.at[2 * j]], add=True)

        @pl.when(2 * j + 2 < _CPW - 1)
        def _():
            pltpu.async_copy(ea_hbm.at[pl.ds(base0 + (2 * j + 2) * _CHUNK, _CHUNK)],
                             erow_a, sem_a)

        wait_rows(erow_b, sem_b)
        pltpu.sync_copy(erow_b, acce_sh.at[dst_v.at[2 * j + 1]], add=True)

        @pl.when(2 * j + 3 < _CPW)
        def _():
            pltpu.async_copy(ea_hbm.at[pl.ds(base0 + (2 * j + 3) * _CHUNK, _CHUNK)],
                             erow_b, sem_b)

        return carry

    pltpu.sync_copy(dst_hbm.at[pl.ds(w * _CPWPAD, _CPWPAD)], dst_v)
    pltpu.async_copy(ea_hbm.at[pl.ds(base0, _CHUNK)], erow_a, sem_a)
    pltpu.async_copy(ea_hbm.at[pl.ds(base0 + _CHUNK, _CHUNK)], erow_b, sem_b)
    lax.fori_loop(0, (_CPW - 1) // 2, pipe, 0)
    pltpu.async_copy(ea_hbm.at[pl.ds(base0 + (_CPW - 1) * _CHUNK, _CHUNK)],
                     erow_a, sem_a).wait()
    pltpu.sync_copy(erow_a, acce_sh.at[dst_v.at[_CPW - 1]], add=True)
    plsc.subcore_barrier()

    pltpu.sync_copy(acce_sh.at[pl.ds(sid * _RPT, _RPT)],
                    oute_hbm.at[cid, pl.ds(sid * _RPT, _RPT)])


_sc_mesh = plsc.VectorSubcoreMesh(core_axis_name="c", subcore_axis_name="s")

_sc_aggr_x = pl.kernel(
    _sc_x_body,
    out_type=jax.ShapeDtypeStruct((_NC, _NPAD, _NV), jnp.float32),
    mesh=_sc_mesh,
    scratch_types=[
        pltpu.VMEM((_STAGES[0][1], _CHUNK), jnp.int32),
        pltpu.VMEM((_STAGES[0][1], _CHUNK), jnp.int32),
        pltpu.VMEM((_CHUNK, _NV), jnp.float32),
        pltpu.VMEM((_CHUNK, _NV), jnp.float32),
        pltpu.VMEM_SHARED((_NPAD, _NV), jnp.float32),
        pltpu.SemaphoreType.DMA,
        pltpu.SemaphoreType.DMA,
    ],
)

_sc_aggr_e = pl.kernel(
    _sc_e_body,
    out_type=jax.ShapeDtypeStruct((_NC, _NPAD, _NE), jnp.float32),
    mesh=_sc_mesh,
    scratch_types=[
        pltpu.VMEM((_CPWPAD, _CHUNK), jnp.int32),
        pltpu.VMEM((_CHUNK, _NE), jnp.float32),
        pltpu.VMEM((_CHUNK, _NE), jnp.float32),
        pltpu.VMEM_SHARED((_NPAD, _NE), jnp.float32),
        pltpu.SemaphoreType.DMA,
        pltpu.SemaphoreType.DMA,
    ],
)


def _tc_body(axp_ref, aep_ref, d_ref, w_ref, rp_ref, h_ref, ps_ref):
    i = pl.program_id(0)
    ax = axp_ref[0] + axp_ref[1]
    ae = aep_ref[0] + aep_ref[1]
    d = d_ref[...]
    f = 1.0 / d.astype(jnp.float32)
    sx = ax * f
    se = ae * f

    sxe = jnp.concatenate([sx, se], axis=1).astype(jnp.bfloat16)

    def body(b, acc):
        m = (d == b + 1).astype(jnp.bfloat16)
        acc = acc + jnp.dot(sxe * m, w_ref[b],
                            preferred_element_type=jnp.float32)
        return acc

    acc = lax.fori_loop(0, _B, body, jnp.zeros((_R, _NV), jnp.float32))
    h = 1.0 / (1.0 + jnp.exp(-acc))
    h_ref[...] = h

    logits = jnp.dot(h, rp_ref[...], preferred_element_type=jnp.float32)
    lane = lax.broadcasted_iota(jnp.int32, (_R, _NE), 1)
    valid = lane < _NOUT
    mx = jnp.max(jnp.where(valid, logits, -1e30), axis=1, keepdims=True)
    e = jnp.where(valid, jnp.exp(logits - mx), 0.0)
    p = e / jnp.sum(e, axis=1, keepdims=True)
    colsum = jnp.sum(p, axis=0, keepdims=True)

    @pl.when(i == 0)
    def _():
        ps_ref[...] = jnp.zeros_like(ps_ref)

    ps_ref[...] += colsum


_tc_step = pl.pallas_call(
    _tc_body,
    grid=(_NBLK,),
    in_specs=[
        pl.BlockSpec((_NC, _R, _NV), lambda i: (0, i, 0)),
        pl.BlockSpec((_NC, _R, _NE), lambda i: (0, i, 0)),
        pl.BlockSpec((_R, 1), lambda i: (i, 0)),
        pl.BlockSpec((_B, _NV + _NE, _NV), lambda i: (0, 0, 0)),
        pl.BlockSpec((_NV, _NE), lambda i: (0, 0)),
    ],
    out_specs=[
        pl.BlockSpec((_R, _NV), lambda i: (i, 0)),
        pl.BlockSpec((1, _NE), lambda i: (0, 0)),
    ],
    out_shape=[
        jax.ShapeDtypeStruct((_N, _NV), jnp.float32),
        jax.ShapeDtypeStruct((1, _NE), jnp.float32),
    ],
)


def kernel(x, edge_index, edge_attr, node_degree, weights, readout_weights):
    def pad_idx(v):
        v3 = v.astype(jnp.int32).reshape(_NW, _CPW, _CHUNK)
        v3 = jnp.pad(v3, ((0, 0), (0, _CPWPAD - _CPW), (0, 0)))
        return v3.reshape(_NCHUNKS, _CHUNK)

    src2 = pad_idx(edge_index[0])
    dst2 = pad_idx(edge_index[1])
    ea = edge_attr
    d = jnp.clip(node_degree, _MIND, _MAXD).astype(jnp.int32).reshape(_N, 1)
    w_all = weights.reshape(_T, _B, _NV + _NE, _NV).astype(jnp.bfloat16)
    rp = jnp.pad(readout_weights.reshape(_T, _NV, _NOUT),
                 ((0, 0), (0, 0), (0, _NE - _NOUT)))

    h = x
    aep = _sc_aggr_e(ea, dst2)
    total = jnp.zeros((_NE,), jnp.float32)
    for t in range(_T):
        axp = _sc_aggr_x(h, src2, dst2)
        h, ps = _tc_step(axp, aep, d, w_all[t], rp[t])
        total = total + ps[0]
    return total[:_NOUT]


# speedup vs baseline: 9.3311x; 1.2347x over previous
import functools

import jax
import jax.numpy as jnp
from jax import lax
from jax.experimental import pallas as pl
from jax.experimental.pallas import tpu as pltpu
from jax.experimental.pallas import tpu_sc as plsc

_N = 10000
_E = 320000
_NV = 128
_NE = 16
_MAXD = 32
_MIND = 1
_T = 4
_NOUT = 10
_B = _MAXD - _MIND + 1

_NC = 2
_NS = 16
_NW = _NC * _NS
_CHUNK = 80
_CPW = 125
_CPWPAD = 128
_STAGES = ((0, 64, 64), (64, 64, 61))
_NCHUNKS = _NW * _CPWPAD
_NPAD = 10112
_RPT = _NPAD // _NS

_R = 2000
_NBLK = _N // _R


_DEPTH = 2


def _zero_acc(buf_v, acc_sh, sid):
    width = buf_v.shape[1]
    def zrow(r, carry):
        for c in range(width // 16):
            buf_v[r, pl.ds(c * 16, 16)] = jnp.zeros((16,), jnp.float32)
        return carry

    lax.fori_loop(0, _CHUNK, zrow, 0)
    for k in range(0, _RPT, _CHUNK):
        rows = min(_CHUNK, _RPT - k)
        pltpu.sync_copy(buf_v.at[pl.ds(0, rows)],
                        acc_sh.at[pl.ds(sid * _RPT + k, rows)])


def _sc_x_body(h_hbm, src_hbm, dst_hbm, outx_hbm,
               src_v, dst_v, rows_a, rows_b, accx_sh, sem_a, sem_b):
    cid = lax.axis_index("c")
    sid = lax.axis_index("s")
    w = sid * _NC + cid

    _zero_acc(rows_a, accx_sh, sid)
    plsc.subcore_barrier()

    def wait_rows(buf, sem):
        pltpu.make_async_copy(h_hbm.at[pl.ds(0, _CHUNK)], buf, sem).wait()

    def pipe(nloc, j, carry):
        wait_rows(rows_a, sem_a)
        pltpu.sync_copy(rows_a, accx_sh.at[dst_v.at[2 * j]], add=True)

        @pl.when(2 * j + 2 < nloc)
        def _():
            pltpu.async_copy(h_hbm.at[src_v.at[2 * j + 2]], rows_a, sem_a)

        wait_rows(rows_b, sem_b)
        pltpu.sync_copy(rows_b, accx_sh.at[dst_v.at[2 * j + 1]], add=True)

        @pl.when(2 * j + 3 < nloc)
        def _():
            pltpu.async_copy(h_hbm.at[src_v.at[2 * j + 3]], rows_b, sem_b)

        return carry

    for off, nstage, nproc in _STAGES:
        base = w * _CPWPAD + off
        npipe = nproc - (nproc % 2)
        pltpu.sync_copy(src_hbm.at[pl.ds(base, nstage)], src_v)
        pltpu.sync_copy(dst_hbm.at[pl.ds(base, nstage)], dst_v)
        pltpu.async_copy(h_hbm.at[src_v.at[0]], rows_a, sem_a)
        pltpu.async_copy(h_hbm.at[src_v.at[1]], rows_b, sem_b)
        lax.fori_loop(0, npipe // 2, functools.partial(pipe, npipe), 0)
        for tail in range(npipe, nproc):
            pltpu.async_copy(h_hbm.at[src_v.at[tail]], rows_a, sem_a).wait()
            pltpu.sync_copy(rows_a, accx_sh.at[dst_v.at[tail]], add=True)
    plsc.subcore_barrier()

    pltpu.sync_copy(accx_sh.at[pl.ds(sid * _RPT, _RPT)],
                    outx_hbm.at[cid, pl.ds(sid * _RPT, _RPT)])


def _sc_e_body(ea_hbm, dst_hbm, oute_hbm, dst_v, erow_a, erow_b, acce_sh,
               sem_a, sem_b):
    cid = lax.axis_index("c")
    sid = lax.axis_index("s")
    w = sid * _NC + cid

    _zero_acc(erow_a, acce_sh, sid)
    plsc.subcore_barrier()

    def wait_rows(buf, sem):
        pltpu.make_async_copy(ea_hbm.at[pl.ds(0, _CHUNK)], buf, sem).wait()

    base0 = w * _CPW * _CHUNK

    def pipe(j, carry):
        wait_rows(erow_a, sem_a)
        pltpu.sync_copy(erow_a, acce_sh.at[dst_v.at[2 * j]], add=True)

        @pl.when(2 * j + 2 < _CPW - 1)
        def _():
            pltpu.async_copy(ea_hbm.at[pl.ds(base0 + (2 * j + 2) * _CHUNK, _CHUNK)],
                             erow_a, sem_a)

        wait_rows(erow_b, sem_b)
        pltpu.sync_copy(erow_b, acce_sh.at[dst_v.at[2 * j + 1]], add=True)

        @pl.when(2 * j + 3 < _CPW)
        def _():
            pltpu.async_copy(ea_hbm.at[pl.ds(base0 + (2 * j + 3) * _CHUNK, _CHUNK)],
                             erow_b, sem_b)

        return carry

    pltpu.sync_copy(dst_hbm.at[pl.ds(w * _CPWPAD, _CPWPAD)], dst_v)
    pltpu.async_copy(ea_hbm.at[pl.ds(base0, _CHUNK)], erow_a, sem_a)
    pltpu.async_copy(ea_hbm.at[pl.ds(base0 + _CHUNK, _CHUNK)], erow_b, sem_b)
    lax.fori_loop(0, (_CPW - 1) // 2, pipe, 0)
    pltpu.async_copy(ea_hbm.at[pl.ds(base0 + (_CPW - 1) * _CHUNK, _CHUNK)],
                     erow_a, sem_a).wait()
    pltpu.sync_copy(erow_a, acce_sh.at[dst_v.at[_CPW - 1]], add=True)
    plsc.subcore_barrier()

    pltpu.sync_copy(acce_sh.at[pl.ds(sid * _RPT, _RPT)],
                    oute_hbm.at[cid, pl.ds(sid * _RPT, _RPT)])


_sc_mesh = plsc.VectorSubcoreMesh(core_axis_name="c", subcore_axis_name="s")

_sc_aggr_x = pl.kernel(
    _sc_x_body,
    out_type=jax.ShapeDtypeStruct((_NC, _NPAD, _NV), jnp.float32),
    mesh=_sc_mesh,
    scratch_types=[
        pltpu.VMEM((_STAGES[0][1], _CHUNK), jnp.int32),
        pltpu.VMEM((_STAGES[0][1], _CHUNK), jnp.int32),
        pltpu.VMEM((_CHUNK, _NV), jnp.float32),
        pltpu.VMEM((_CHUNK, _NV), jnp.float32),
        pltpu.VMEM_SHARED((_NPAD, _NV), jnp.float32),
        pltpu.SemaphoreType.DMA,
        pltpu.SemaphoreType.DMA,
    ],
)

_sc_aggr_e = pl.kernel(
    _sc_e_body,
    out_type=jax.ShapeDtypeStruct((_NC, _NPAD, _NE), jnp.float32),
    mesh=_sc_mesh,
    scratch_types=[
        pltpu.VMEM((_CPWPAD, _CHUNK), jnp.int32),
        pltpu.VMEM((_CHUNK, _NE), jnp.float32),
        pltpu.VMEM((_CHUNK, _NE), jnp.float32),
        pltpu.VMEM_SHARED((_NPAD, _NE), jnp.float32),
        pltpu.SemaphoreType.DMA,
        pltpu.SemaphoreType.DMA,
    ],
)


def _tc_body(axp_ref, aep_ref, d_ref, w_ref, rp_ref, h_ref, ps_ref):
    i = pl.program_id(0)
    ax = axp_ref[0] + axp_ref[1]
    ae = aep_ref[0] + aep_ref[1]
    d = d_ref[...]
    f = 1.0 / d.astype(jnp.float32)
    sx = ax * f
    se = ae * f

    sxe = jnp.concatenate([sx, se], axis=1).astype(jnp.bfloat16)

    def body(b, acc):
        m = (d == b + 1).astype(jnp.bfloat16)
        acc = acc + jnp.dot(sxe * m, w_ref[b],
                            preferred_element_type=jnp.float32)
        return acc

    acc = jnp.zeros((_R, _NV), jnp.float32)
    for b in range(_B):
        acc = body(b, acc)
    h = 1.0 / (1.0 + jnp.exp(-acc))
    h_ref[...] = h

    logits = jnp.dot(h, rp_ref[...], preferred_element_type=jnp.float32)
    lane = lax.broadcasted_iota(jnp.int32, (_R, _NE), 1)
    valid = lane < _NOUT
    mx = jnp.max(jnp.where(valid, logits, -1e30), axis=1, keepdims=True)
    e = jnp.where(valid, jnp.exp(logits - mx), 0.0)
    p = e / jnp.sum(e, axis=1, keepdims=True)
    colsum = jnp.sum(p, axis=0, keepdims=True)

    @pl.when(i == 0)
    def _():
        ps_ref[...] = jnp.zeros_like(ps_ref)

    ps_ref[...] += colsum


_tc_step = pl.pallas_call(
    _tc_body,
    grid=(_NBLK,),
    in_specs=[
        pl.BlockSpec((_NC, _R, _NV), lambda i: (0, i, 0)),
        pl.BlockSpec((_NC, _R, _NE), lambda i: (0, i, 0)),
        pl.BlockSpec((_R, 1), lambda i: (i, 0)),
        pl.BlockSpec((_B, _NV + _NE, _NV), lambda i: (0, 0, 0)),
        pl.BlockSpec((_NV, _NE), lambda i: (0, 0)),
    ],
    out_specs=[
        pl.BlockSpec((_R, _NV), lambda i: (i, 0)),
        pl.BlockSpec((1, _NE), lambda i: (0, 0)),
    ],
    out_shape=[
        jax.ShapeDtypeStruct((_N, _NV), jnp.float32),
        jax.ShapeDtypeStruct((1, _NE), jnp.float32),
    ],
)


def kernel(x, edge_index, edge_attr, node_degree, weights, readout_weights):
    def pad_idx(v):
        v3 = v.astype(jnp.int32).reshape(_NW, _CPW, _CHUNK)
        v3 = jnp.pad(v3, ((0, 0), (0, _CPWPAD - _CPW), (0, 0)))
        return v3.reshape(_NCHUNKS, _CHUNK)

    src2 = pad_idx(edge_index[0])
    dst2 = pad_idx(edge_index[1])
    ea = edge_attr
    d = jnp.clip(node_degree, _MIND, _MAXD).astype(jnp.int32).reshape(_N, 1)
    w_all = weights.reshape(_T, _B, _NV + _NE, _NV).astype(jnp.bfloat16)
    rp = jnp.pad(readout_weights.reshape(_T, _NV, _NOUT),
                 ((0, 0), (0, 0), (0, _NE - _NOUT)))

    h = x
    aep = _sc_aggr_e(ea, dst2)
    total = jnp.zeros((_NE,), jnp.float32)
    for t in range(_T):
        axp = _sc_aggr_x(h, src2, dst2)
        h, ps = _tc_step(axp, aep, d, w_all[t], rp[t])
        total = total + ps[0]
    return total[:_NOUT]
